# baseline (device time: 50403 ns/iter reference)
import jax
import jax.numpy as jnp
from jax import lax
from jax.experimental import pallas as pl
from jax.experimental.pallas import tpu as pltpu

N_DEV = 16
N_PEERS = N_DEV - 1
N_LAYERS = 3
GROUPS = ((0, 8), (8, 15))


def kernel(x, Win0, Wout0, Win1, Wout1, Win2, Wout2):
    b, d_shard = x.shape
    h_dim = Win0.shape[1]
    rows = b // N_DEV

    def body(x_ref, win0_ref, wout0_ref, win1_ref, wout1_ref, win2_ref,
             wout2_ref, out_ref, acc_ref, rs_recv, h_slot,
             rs_ssem, rs_rsem, ag_ssem, ag_rsem):
        my_i = lax.axis_index("i")
        wins = [win0_ref, win1_ref, win2_ref]
        wouts = [wout0_ref, wout1_ref, wout2_ref]
        all_rdmas = []

        def remote_copy(src, dst, ssem, rsem, target):
            r = pltpu.make_async_remote_copy(
                src_ref=src, dst_ref=dst, send_sem=ssem, recv_sem=rsem,
                device_id=(target,), device_id_type=pl.DeviceIdType.MESH,
            )
            r.start()
            all_rdmas.append(r)

        acc_ref[0] = jnp.dot(
            x_ref[...], win0_ref[...], preferred_element_type=jnp.float32
        ).astype(jnp.bfloat16)
        for j in range(N_PEERS):
            p = (my_i + 1 + j) % N_DEV
            remote_copy(acc_ref.at[0, pl.ds(rows * p, rows), :],
                        rs_recv.at[0, j], rs_ssem.at[0, j],
                        rs_rsem.at[0, j], p)

        own_f32 = None
        for l in range(N_LAYERS):
            if l == 0:
                red = acc_ref[0, pl.ds(rows * my_i, rows), :].astype(
                    jnp.float32
                )
            else:
                red = own_f32
            for j in range(N_PEERS):
                w = pltpu.make_async_remote_copy(
                    src_ref=acc_ref.at[l, pl.ds(0, rows), :],
                    dst_ref=rs_recv.at[l, j],
                    send_sem=rs_ssem.at[l, j],
                    recv_sem=rs_rsem.at[l, j],
                    device_id=(my_i,),
                    device_id_type=pl.DeviceIdType.MESH,
                )
                w.wait_recv()
                red = red + rs_recv[l, j].astype(jnp.float32)
            h_chunk = jnp.maximum(red, 0.0)
            h_slot[l, N_DEV - 1] = h_chunk.astype(jnp.bfloat16)

            ag_rdmas = []
            for j in range(N_PEERS):
                p = (my_i + 1 + j) % N_DEV
                r = pltpu.make_async_remote_copy(
                    src_ref=h_slot.at[l, N_DEV - 1],
                    dst_ref=h_slot.at[l, j],
                    send_sem=ag_ssem.at[l, j],
                    recv_sem=ag_rsem.at[l, j],
                    device_id=(p,),
                    device_id_type=pl.DeviceIdType.MESH,
                )
                r.start()
                ag_rdmas.append(r)
                all_rdmas.append(r)

            if l < N_LAYERS - 1:
                y_own = jnp.dot(
                    h_slot[l, N_DEV - 1], wouts[l][...],
                    preferred_element_type=jnp.float32,
                )
                pa_own = jnp.dot(
                    y_own, wins[l + 1][...],
                    preferred_element_type=jnp.float32,
                )
                acc_ref[l + 1, pl.ds(rows * N_PEERS, rows), :] = (
                    pa_own.astype(jnp.bfloat16)
                )
                remote_copy(acc_ref.at[l + 1, pl.ds(rows * N_PEERS, rows), :],
                            rs_recv.at[l + 1, 7],
                            rs_ssem.at[l + 1, N_PEERS],
                            rs_rsem.at[l + 1, 7],
                            (my_i + 8) % N_DEV)
            else:
                y_own = jnp.dot(
                    h_slot[l, N_DEV - 1], wouts[l][...],
                    preferred_element_type=jnp.float32,
                )
                out_ref[pl.ds(rows * my_i, rows), :] = y_own

            for lo, hi in GROUPS:
                for j in range(lo, hi):
                    ag_rdmas[j].wait_recv()
                hh = h_slot[l, lo:hi].reshape((hi - lo) * rows, h_dim)
                y = jnp.dot(
                    hh, wouts[l][...], preferred_element_type=jnp.float32
                )
                if l < N_LAYERS - 1:
                    pa = jnp.dot(
                        y, wins[l + 1][...],
                        preferred_element_type=jnp.float32,
                    )
                    acc_ref[l + 1, pl.ds(rows * lo, (hi - lo) * rows), :] = (
                        pa.astype(jnp.bfloat16)
                    )
                    for j in range(lo, hi):
                        if j == 7:
                            continue
                        remote_copy(
                            acc_ref.at[l + 1, pl.ds(rows * j, rows), :],
                            rs_recv.at[l + 1, (6 - j) % N_DEV],
                            rs_ssem.at[l + 1, j],
                            rs_rsem.at[l + 1, (6 - j) % N_DEV],
                            (my_i + 7 - j) % N_DEV,
                        )
                    if lo <= 7 < hi:
                        own_f32 = pa[(7 - lo) * rows:(8 - lo) * rows, :]
                else:
                    for j in range(lo, hi):
                        i_org = (my_i - 1 - j) % N_DEV
                        out_ref[pl.ds(rows * i_org, rows), :] = y[
                            (j - lo) * rows:(j - lo + 1) * rows, :
                        ]

        for r in all_rdmas:
            r.wait_send()

    return pl.pallas_call(
        body,
        out_shape=jax.ShapeDtypeStruct((b, d_shard), jnp.float32),
        in_specs=[pl.BlockSpec(memory_space=pltpu.VMEM)] * 7,
        out_specs=pl.BlockSpec(memory_space=pltpu.VMEM),
        scratch_shapes=[
            pltpu.VMEM((N_LAYERS, b, h_dim), jnp.bfloat16),
            pltpu.VMEM((N_LAYERS, N_PEERS, rows, h_dim), jnp.bfloat16),
            pltpu.VMEM((N_LAYERS, N_DEV, rows, h_dim), jnp.bfloat16),
            pltpu.SemaphoreType.DMA((N_LAYERS, N_DEV)),
            pltpu.SemaphoreType.DMA((N_LAYERS, N_PEERS)),
            pltpu.SemaphoreType.DMA((N_LAYERS, N_PEERS)),
            pltpu.SemaphoreType.DMA((N_LAYERS, N_PEERS)),
        ],
    )(x, Win0, Wout0, Win1, Wout1, Win2, Wout2)
